# baseline (speedup 1.0000x reference)
import functools

import jax
import jax.numpy as jnp
from jax import lax
from jax.experimental import pallas as pl
from jax.experimental.pallas import tpu as pltpu

E = 8
TOPK = 2
D = 1024
F = 4096
B = 1
S = 2048

BLK = 256
R_PAD = S * TOPK + E * BLK
NB = R_PAD // BLK
FT = 512
NF = F // FT

INTERPRET = False


def _router_body(x_ref, wr_ref, br_ref, sel_ref, w_ref, rank_ref, cnt_ref,
                 base_ref):
    i = pl.program_id(0)

    @pl.when(i == 0)
    def _():
        base_ref[...] = jnp.zeros_like(base_ref)

    x = x_ref[...]
    logits = jnp.dot(x, wr_ref[...],
                     preferred_element_type=jnp.float32) + br_ref[...]
    m = jnp.max(logits, axis=-1, keepdims=True)
    unnorm = jnp.exp(logits - m)
    probs = unnorm / jnp.sum(unnorm, axis=-1, keepdims=True)

    iota_e = lax.broadcasted_iota(jnp.int32, probs.shape, 1)
    m0 = jnp.max(probs, axis=-1, keepdims=True)
    i0 = jnp.min(jnp.where(probs == m0, iota_e, E), axis=-1, keepdims=True)
    probs_m = jnp.where(iota_e == i0, -1.0, probs)
    m1 = jnp.max(probs_m, axis=-1, keepdims=True)
    i1 = jnp.min(jnp.where(probs_m == m1, iota_e, E), axis=-1, keepdims=True)
    ssum = m0 + m1
    w0 = m0 / ssum
    w1 = m1 / ssum

    maskf = ((iota_e == i0) | (iota_e == i1)).astype(jnp.float32)

    sb = probs.shape[0]
    r_iota = lax.broadcasted_iota(jnp.int32, (sb, sb), 0)
    c_iota = lax.broadcasted_iota(jnp.int32, (sb, sb), 1)
    tril = (r_iota > c_iota).astype(jnp.float32)
    local_excl = jnp.dot(tril, maskf, preferred_element_type=jnp.float32)
    rank_f = base_ref[...] + local_excl
    rank_ref[...] = rank_f.astype(jnp.int32)

    sel_ref[...] = jnp.where(iota_e == 0, i0,
                             jnp.where(iota_e == 1, i1, 0)).astype(jnp.int32)
    w_ref[...] = jnp.where(iota_e == 0, w0,
                           jnp.where(iota_e == 1, w1, 0.0))

    base_ref[...] = base_ref[...] + jnp.sum(maskf, axis=0, keepdims=True)

    @pl.when(i == pl.num_programs(0) - 1)
    def _():
        cnt = base_ref[...].astype(jnp.int32)
        cnt_ref[...] = jnp.concatenate(
            [cnt, jnp.zeros((1, 16 - E), jnp.int32)], axis=1)


def _router(x, wr, br):
    sb = 512
    grid = (S // sb,)
    return pl.pallas_call(
        _router_body,
        grid=grid,
        in_specs=[
            pl.BlockSpec((sb, D), lambda i: (i, 0)),
            pl.BlockSpec((D, E), lambda i: (0, 0)),
            pl.BlockSpec((1, E), lambda i: (0, 0)),
        ],
        out_specs=[
            pl.BlockSpec((sb, E), lambda i: (i, 0)),
            pl.BlockSpec((sb, E), lambda i: (i, 0)),
            pl.BlockSpec((sb, E), lambda i: (i, 0)),
            pl.BlockSpec((1, 16), lambda i: (0, 0)),
        ],
        out_shape=[
            jax.ShapeDtypeStruct((S, E), jnp.int32),
            jax.ShapeDtypeStruct((S, E), jnp.float32),
            jax.ShapeDtypeStruct((S, E), jnp.int32),
            jax.ShapeDtypeStruct((1, 16), jnp.int32),
        ],
        scratch_shapes=[pltpu.VMEM((1, E), jnp.float32)],
        interpret=INTERPRET,
    )(x, wr, br)


def _gmm_body(be_ref, xs_ref, w1_ref, b1_ref, w2_ref, b2_ref, y_ref):
    f = pl.program_id(1)

    @pl.when(f == 0)
    def _():
        y_ref[...] = jnp.broadcast_to(b2_ref[0], y_ref.shape)

    h = jnp.dot(xs_ref[...], w1_ref[0],
                preferred_element_type=jnp.float32) + b1_ref[0]
    h = jnp.maximum(h, 0.0)
    y_ref[...] += jnp.dot(h, w2_ref[0], preferred_element_type=jnp.float32)


def _grouped_matmul(xs, w1, b1, w2, b2, block_expert):
    grid = (NB, NF)
    return pl.pallas_call(
        _gmm_body,
        grid_spec=pltpu.PrefetchScalarGridSpec(
            num_scalar_prefetch=1,
            grid=grid,
            in_specs=[
                pl.BlockSpec((BLK, D), lambda i, f, be: (i, 0)),
                pl.BlockSpec((1, D, FT), lambda i, f, be: (be[i], 0, f)),
                pl.BlockSpec((1, 1, FT), lambda i, f, be: (be[i], 0, f)),
                pl.BlockSpec((1, FT, D), lambda i, f, be: (be[i], f, 0)),
                pl.BlockSpec((1, 1, D), lambda i, f, be: (be[i], 0, 0)),
            ],
            out_specs=pl.BlockSpec((BLK, D), lambda i, f, be: (i, 0)),
        ),
        out_shape=jax.ShapeDtypeStruct((R_PAD, D), jnp.float32),
        interpret=INTERPRET,
    )(block_expert, xs, w1, b1.reshape(E, 1, F), w2, b2.reshape(E, 1, D))


def _dispatch(sel, rank, counts):
    cnt = counts[0, :E]
    blocks_e = (cnt + BLK - 1) // BLK
    cum_inc = jnp.cumsum(blocks_e)
    off_rows = (cum_inc - blocks_e) * BLK
    sel2 = sel[:, :TOPK]
    rank2 = jnp.take_along_axis(rank, sel2, axis=1)
    p = off_rows[sel2] + rank2
    p_flat = p.reshape(-1)
    tok = jnp.arange(S * TOPK, dtype=jnp.int32) // TOPK
    token_of_row = jnp.zeros((R_PAD,), jnp.int32).at[p_flat].set(tok)
    be = jnp.searchsorted(cum_inc, jnp.arange(NB, dtype=jnp.int32),
                          side='right').astype(jnp.int32)
    be = jnp.minimum(be, E - 1)
    return p, token_of_row, be


def kernel(input_emb, Wr, br, W1, b1, W2, b2):
    x = input_emb.reshape(S, D)
    sel, w, rank, counts = _router(x, Wr, br.reshape(1, E))
    p, token_of_row, block_expert = _dispatch(sel, rank, counts)
    xs = x[token_of_row]
    y = _grouped_matmul(xs, W1, b1, W2, b2, block_expert)
    out = w[:, 0:1] * y[p[:, 0]] + w[:, 1:2] * y[p[:, 1]]
    return out.reshape(B, S, D)


# speedup vs baseline: 1.3477x; 1.3477x over previous
import functools

import jax
import jax.numpy as jnp
from jax import lax
from jax.experimental import pallas as pl
from jax.experimental.pallas import tpu as pltpu

E = 8
TOPK = 2
D = 1024
F = 4096
B = 1
S = 2048

BLK = 512
R_PAD = S * TOPK + E * BLK
NB = R_PAD // BLK
FT = 512
NF = F // FT

INTERPRET = False


def _router_body(x_ref, wr_ref, br_ref, sel_ref, w_ref, rank_ref, cnt_ref,
                 base_ref):
    i = pl.program_id(0)

    @pl.when(i == 0)
    def _():
        base_ref[...] = jnp.zeros_like(base_ref)

    x = x_ref[...]
    logits = jnp.dot(x, wr_ref[...],
                     preferred_element_type=jnp.float32) + br_ref[...]
    m = jnp.max(logits, axis=-1, keepdims=True)
    unnorm = jnp.exp(logits - m)
    probs = unnorm / jnp.sum(unnorm, axis=-1, keepdims=True)

    iota_e = lax.broadcasted_iota(jnp.int32, probs.shape, 1)
    m0 = jnp.max(probs, axis=-1, keepdims=True)
    i0 = jnp.min(jnp.where(probs == m0, iota_e, E), axis=-1, keepdims=True)
    probs_m = jnp.where(iota_e == i0, -1.0, probs)
    m1 = jnp.max(probs_m, axis=-1, keepdims=True)
    i1 = jnp.min(jnp.where(probs_m == m1, iota_e, E), axis=-1, keepdims=True)
    ssum = m0 + m1
    w0 = m0 / ssum
    w1 = m1 / ssum

    maskf = ((iota_e == i0) | (iota_e == i1)).astype(jnp.float32)

    sb = probs.shape[0]
    r_iota = lax.broadcasted_iota(jnp.int32, (sb, sb), 0)
    c_iota = lax.broadcasted_iota(jnp.int32, (sb, sb), 1)
    tril = (r_iota > c_iota).astype(jnp.float32)
    local_excl = jnp.dot(tril, maskf, preferred_element_type=jnp.float32)
    rank_f = base_ref[...] + local_excl
    rank_ref[...] = rank_f.astype(jnp.int32)

    sel_ref[...] = jnp.where(iota_e == 0, i0,
                             jnp.where(iota_e == 1, i1, 0)).astype(jnp.int32)
    w_ref[...] = jnp.where(iota_e == 0, w0,
                           jnp.where(iota_e == 1, w1, 0.0))

    base_ref[...] = base_ref[...] + jnp.sum(maskf, axis=0, keepdims=True)

    @pl.when(i == pl.num_programs(0) - 1)
    def _():
        cnt = base_ref[...].astype(jnp.int32)
        cnt_ref[...] = jnp.concatenate(
            [cnt, jnp.zeros((1, 16 - E), jnp.int32)], axis=1)


def _router(x, wr, br):
    sb = 512
    grid = (S // sb,)
    return pl.pallas_call(
        _router_body,
        grid=grid,
        in_specs=[
            pl.BlockSpec((sb, D), lambda i: (i, 0)),
            pl.BlockSpec((D, E), lambda i: (0, 0)),
            pl.BlockSpec((1, E), lambda i: (0, 0)),
        ],
        out_specs=[
            pl.BlockSpec((sb, E), lambda i: (i, 0)),
            pl.BlockSpec((sb, E), lambda i: (i, 0)),
            pl.BlockSpec((sb, E), lambda i: (i, 0)),
            pl.BlockSpec((1, 16), lambda i: (0, 0)),
        ],
        out_shape=[
            jax.ShapeDtypeStruct((S, E), jnp.int32),
            jax.ShapeDtypeStruct((S, E), jnp.float32),
            jax.ShapeDtypeStruct((S, E), jnp.int32),
            jax.ShapeDtypeStruct((1, 16), jnp.int32),
        ],
        scratch_shapes=[pltpu.VMEM((1, E), jnp.float32)],
        interpret=INTERPRET,
    )(x, wr, br)


def _gmm_body(bex_ref, lv_ref, valid_ref, xs_ref, w1_ref, b1_ref, w2_ref,
              b2_ref, y_ref):
    i = pl.program_id(0)
    f = pl.program_id(1)
    is_valid = valid_ref[i] == 1

    @pl.when(jnp.logical_and(f == 0, is_valid))
    def _():
        y_ref[...] = jnp.broadcast_to(b2_ref[0], y_ref.shape)

    @pl.when(is_valid)
    def _():
        x16 = xs_ref[...].astype(jnp.bfloat16)
        w116 = w1_ref[0].astype(jnp.bfloat16)
        h = jnp.dot(x16, w116, preferred_element_type=jnp.float32) + b1_ref[0]
        h = jnp.maximum(h, 0.0)
        w216 = w2_ref[0].astype(jnp.bfloat16)
        y_ref[...] += jnp.dot(h.astype(jnp.bfloat16), w216,
                              preferred_element_type=jnp.float32)


def _grouped_matmul(xs, w1, b1, w2, b2, bex, lv, valid):
    grid = (NB, NF)

    def w_idx(i, f, bex, lv, valid):
        f_eff = jnp.where(valid[i] == 1, f, NF - 1)
        return (bex[i], 0, f_eff)

    def w2_idx(i, f, bex, lv, valid):
        f_eff = jnp.where(valid[i] == 1, f, NF - 1)
        return (bex[i], f_eff, 0)

    return pl.pallas_call(
        _gmm_body,
        grid_spec=pltpu.PrefetchScalarGridSpec(
            num_scalar_prefetch=3,
            grid=grid,
            in_specs=[
                pl.BlockSpec((BLK, D), lambda i, f, bex, lv, valid: (lv[i], 0)),
                pl.BlockSpec((1, D, FT), w_idx),
                pl.BlockSpec((1, 1, FT), w_idx),
                pl.BlockSpec((1, FT, D), w2_idx),
                pl.BlockSpec((1, 1, D),
                             lambda i, f, bex, lv, valid: (bex[i], 0, 0)),
            ],
            out_specs=pl.BlockSpec((BLK, D),
                                   lambda i, f, bex, lv, valid: (lv[i], 0)),
        ),
        out_shape=jax.ShapeDtypeStruct((R_PAD, D), jnp.float32),
        interpret=INTERPRET,
    )(bex, lv, valid, xs, w1, b1.reshape(E, 1, F), w2, b2.reshape(E, 1, D))


def _dispatch(sel, rank, counts):
    cnt = counts[0, :E]
    blocks_e = (cnt + BLK - 1) // BLK
    cum_inc = jnp.cumsum(blocks_e)
    off_rows = (cum_inc - blocks_e) * BLK
    sel2 = sel[:, :TOPK]
    rank2 = jnp.take_along_axis(rank, sel2, axis=1)
    p = off_rows[sel2] + rank2
    p_flat = p.reshape(-1)
    tok = jnp.arange(S * TOPK, dtype=jnp.int32) // TOPK
    token_of_row = jnp.zeros((R_PAD,), jnp.int32).at[p_flat].set(tok)
    total_blocks = cum_inc[E - 1]
    barange = jnp.arange(NB, dtype=jnp.int32)
    valid = (barange < total_blocks).astype(jnp.int32)
    bex = jnp.searchsorted(
        cum_inc, jnp.minimum(barange, total_blocks - 1),
        side='right').astype(jnp.int32)
    bex = jnp.minimum(bex, E - 1)
    lv = jnp.where(valid == 1, barange, total_blocks - 1)
    return p, token_of_row, bex, lv, valid


def kernel(input_emb, Wr, br, W1, b1, W2, b2):
    x = input_emb.reshape(S, D)
    sel, w, rank, counts = _router(x, Wr, br.reshape(1, E))
    p, token_of_row, bex, lv, valid = _dispatch(sel, rank, counts)
    xs = x[token_of_row]
    y = _grouped_matmul(xs, W1, b1, W2, b2, bex, lv, valid)
    out = w[:, 0:1] * y[p[:, 0]] + w[:, 1:2] * y[p[:, 1]]
    return out.reshape(B, S, D)
